# Initial kernel scaffold; baseline (speedup 1.0000x reference)
import jax
import jax.numpy as jnp
from jax.experimental import pallas as pl


def kernel(ranks, rerank_dba_final, res_top1000_dba, ranks_trans_1000_pre, x_dba):
    raise NotImplementedError("write your pallas kernel here")


# speedup vs baseline: 4.6205x; 4.6205x over previous
import jax
import jax.numpy as jnp
from jax.experimental import pallas as pl
from jax.experimental.pallas import tpu as pltpu

M = 400
K = 10
Q = 256
N_DB = 100000
D = 512
QB = 8
RB = 400


def _head_body(pre_ref, res_ref, rrv_ref, x_ref, head_ref):
    pre = pre_ref[...]
    r = res_ref[...]
    rrv = rrv_ref[...]
    x = x_ref[...]

    prek = pre[:, :K]
    j_k = jax.lax.broadcasted_iota(jnp.int32, (QB, K, M), 2)
    maskj = jnp.any(prek[:, :, None] == j_k, axis=1)
    pen = (maskj.astype(jnp.float32) - jnp.float32(1.0)) * jnp.float32(1e30)
    xm = x + pen[:, :, None]
    v = jnp.max(xm, axis=1)

    x2 = x.reshape(QB * M, D)
    vt = jnp.transpose(v, (1, 0))
    s_big = jax.lax.dot_general(
        x2, vt, (((1,), (0,)), ((), ())),
        preferred_element_type=jnp.float32)
    s3 = s_big.reshape(QB, M, QB)
    eye = (jax.lax.broadcasted_iota(jnp.int32, (QB, 1, QB), 0)
           == jax.lax.broadcasted_iota(jnp.int32, (QB, 1, QB), 2)
           ).astype(jnp.float32)
    s_all = jnp.sum(s3 * eye, axis=2)

    m_i = jax.lax.broadcasted_iota(jnp.int32, (QB, M, M), 1)
    j_i = jax.lax.broadcasted_iota(jnp.int32, (QB, M, M), 2)

    a = r[:, :, None]
    b = r[:, None, :]
    rank2 = jnp.sum(((b > a) | ((b == a) & (j_i < m_i))).astype(jnp.int32),
                    axis=2)
    oh = (rank2[:, :, None] == j_i).astype(jnp.float32)
    sortedv = jnp.sum(oh * r[:, :, None], axis=1)

    ohp = (pre[:, :, None] == j_i).astype(jnp.float32)
    s_perm = jnp.sum(ohp * s_all[:, None, :], axis=2)

    res = (sortedv + s_perm) * jnp.float32(0.5)

    a2 = res[:, :, None]
    b2 = res[:, None, :]
    rank = jnp.sum(((b2 > a2) | ((b2 == a2) & (j_i < m_i))).astype(jnp.int32),
                   axis=2)
    oh2 = (rank[:, :, None] == j_i).astype(jnp.float32)
    reord = jnp.sum(oh2 * rrv.astype(jnp.float32)[:, :, None], axis=1)
    head_ref[...] = reord.astype(jnp.int32)


def _asm_body(head_ref, ranks_ref, out_ref):
    i = pl.program_id(0)

    @pl.when(i == 0)
    def _():
        out_ref[...] = jnp.transpose(head_ref[...], (1, 0))

    @pl.when(i > 0)
    def _():
        out_ref[...] = ranks_ref[...]


def kernel(ranks, rerank_dba_final, res_top1000_dba, ranks_trans_1000_pre,
           x_dba):
    head = pl.pallas_call(
        _head_body,
        grid=(Q // QB,),
        in_specs=[
            pl.BlockSpec((QB, M), lambda i: (i, 0)),
            pl.BlockSpec((QB, M), lambda i: (i, 0)),
            pl.BlockSpec((QB, M), lambda i: (i, 0)),
            pl.BlockSpec((QB, M, D), lambda i: (i, 0, 0)),
        ],
        out_specs=pl.BlockSpec((QB, M), lambda i: (i, 0)),
        out_shape=jax.ShapeDtypeStruct((Q, M), jnp.int32),
    )(ranks_trans_1000_pre, res_top1000_dba, rerank_dba_final, x_dba)

    out = pl.pallas_call(
        _asm_body,
        grid=(N_DB // RB,),
        in_specs=[
            pl.BlockSpec((Q, M), lambda i: (0, 0)),
            pl.BlockSpec((RB, Q), lambda i: (i, 0)),
        ],
        out_specs=pl.BlockSpec((RB, Q), lambda i: (i, 0)),
        out_shape=jax.ShapeDtypeStruct((N_DB, Q), jnp.int32),
    )(head, ranks)
    return out


# speedup vs baseline: 4.8592x; 1.0517x over previous
import jax
import jax.numpy as jnp
from jax.experimental import pallas as pl
from jax.experimental.pallas import tpu as pltpu

M = 400
K = 10
Q = 256
N_DB = 100000
D = 512
QB = 8
RB = 400


def _gather_lanes(src, idx):
    srcp = jnp.concatenate(
        [src, jnp.zeros((QB, 512 - M), src.dtype)], axis=1)
    acc = None
    for c in range(4):
        chunk = srcp[:, c * 128:(c + 1) * 128]
        loc = idx - jnp.int32(c * 128)
        sel = (loc >= 0) & (loc < 128)
        g = jnp.take_along_axis(chunk, jnp.clip(loc, 0, 127), axis=1)
        part = jnp.where(sel, g, jnp.zeros((), src.dtype))
        acc = part if acc is None else acc + part
    return acc


def _head_body(prek_ref, pre_ref, res_ref, rrv_ref, x_ref, head_ref):
    i = pl.program_id(0)
    pre = pre_ref[...]
    r = res_ref[...]
    rrv = rrv_ref[...]
    x = x_ref[...]

    rows = []
    for q in range(QB):
        base = i * QB + q
        vq = x_ref[pl.ds(q, 1), pl.ds(prek_ref[base, 0], 1), :]
        for k in range(1, K):
            vq = jnp.maximum(
                vq, x_ref[pl.ds(q, 1), pl.ds(prek_ref[base, k], 1), :])
        rows.append(vq.reshape(1, D))
    v = jnp.concatenate(rows, axis=0)

    x2 = x.reshape(QB * M, D)
    vt = jnp.transpose(v, (1, 0))
    s_big = jax.lax.dot_general(
        x2, vt, (((1,), (0,)), ((), ())),
        preferred_element_type=jnp.float32)
    s3 = s_big.reshape(QB, M, QB)
    eye = (jax.lax.broadcasted_iota(jnp.int32, (QB, 1, QB), 0)
           == jax.lax.broadcasted_iota(jnp.int32, (QB, 1, QB), 2)
           ).astype(jnp.float32)
    s_all = jnp.sum(s3 * eye, axis=2)

    m_i = jax.lax.broadcasted_iota(jnp.int32, (QB, M, M), 1)
    j_i = jax.lax.broadcasted_iota(jnp.int32, (QB, M, M), 2)

    a = r[:, :, None]
    b = r[:, None, :]
    rank2 = jnp.sum(((b > a) | ((b == a) & (j_i < m_i))).astype(jnp.int32),
                    axis=2)
    oh = (rank2[:, :, None] == j_i).astype(jnp.float32)
    sortedv = jnp.sum(oh * r[:, :, None], axis=1)

    ohp = (pre[:, :, None] == j_i).astype(jnp.float32)
    s_perm = jnp.sum(ohp * s_all[:, None, :], axis=2)

    res = (sortedv + s_perm) * jnp.float32(0.5)

    a2 = res[:, :, None]
    b2 = res[:, None, :]
    rank = jnp.sum(((b2 > a2) | ((b2 == a2) & (j_i < m_i))).astype(jnp.int32),
                   axis=2)
    oh2 = (rank[:, :, None] == j_i).astype(jnp.float32)
    reord = jnp.sum(oh2 * rrv.astype(jnp.float32)[:, :, None], axis=1)
    head_ref[...] = reord.astype(jnp.int32)


def _asm_body(head_ref, ranks_ref, out_ref):
    i = pl.program_id(0)

    @pl.when(i == 0)
    def _():
        out_ref[...] = jnp.transpose(head_ref[...], (1, 0))

    @pl.when(i > 0)
    def _():
        out_ref[...] = ranks_ref[...]


def kernel(ranks, rerank_dba_final, res_top1000_dba, ranks_trans_1000_pre,
           x_dba):
    prek = ranks_trans_1000_pre[:, :K]
    head = pl.pallas_call(
        _head_body,
        grid=(Q // QB,),
        in_specs=[
            pl.BlockSpec(memory_space=pltpu.SMEM),
            pl.BlockSpec((QB, M), lambda i: (i, 0)),
            pl.BlockSpec((QB, M), lambda i: (i, 0)),
            pl.BlockSpec((QB, M), lambda i: (i, 0)),
            pl.BlockSpec((QB, M, D), lambda i: (i, 0, 0)),
        ],
        out_specs=pl.BlockSpec((QB, M), lambda i: (i, 0)),
        out_shape=jax.ShapeDtypeStruct((Q, M), jnp.int32),
    )(prek, ranks_trans_1000_pre, res_top1000_dba, rerank_dba_final, x_dba)

    out = pl.pallas_call(
        _asm_body,
        grid=(N_DB // RB,),
        in_specs=[
            pl.BlockSpec((Q, M), lambda i: (0, 0)),
            pl.BlockSpec((RB, Q), lambda i: (i, 0)),
        ],
        out_specs=pl.BlockSpec((RB, Q), lambda i: (i, 0)),
        out_shape=jax.ShapeDtypeStruct((N_DB, Q), jnp.int32),
    )(head, ranks)
    return out


# speedup vs baseline: 7.1250x; 1.4663x over previous
import functools

import jax
import jax.numpy as jnp
from jax.experimental import pallas as pl
from jax.experimental.pallas import tpu as pltpu
from jax.experimental.pallas import tpu_sc as plsc

M = 400
K = 10
Q = 256
N_DB = 100000
D = 512
QB = 8
RB = 400


def _gather_lanes(src, idx):
    srcp = jnp.concatenate(
        [src, jnp.zeros((QB, 512 - M), src.dtype)], axis=1)
    acc = None
    for c in range(4):
        chunk = srcp[:, c * 128:(c + 1) * 128]
        loc = idx - jnp.int32(c * 128)
        sel = (loc >= 0) & (loc < 128)
        g = jnp.take_along_axis(chunk, jnp.clip(loc, 0, 127), axis=1)
        part = jnp.where(sel, g, jnp.zeros((), src.dtype))
        acc = part if acc is None else acc + part
    return acc


def _head_body(prek_ref, pre_ref, res_ref, rrv_ref, x_ref, head_ref):
    i = pl.program_id(0)
    pre = pre_ref[...]
    r = res_ref[...]
    rrv = rrv_ref[...]
    x = x_ref[...]

    rows = []
    for q in range(QB):
        base = i * QB + q
        vq = x_ref[pl.ds(q, 1), pl.ds(prek_ref[base, 0], 1), :]
        for k in range(1, K):
            vq = jnp.maximum(
                vq, x_ref[pl.ds(q, 1), pl.ds(prek_ref[base, k], 1), :])
        rows.append(vq.reshape(1, D))
    v = jnp.concatenate(rows, axis=0)

    x2 = x.reshape(QB * M, D)
    vt = jnp.transpose(v, (1, 0))
    s_big = jax.lax.dot_general(
        x2, vt, (((1,), (0,)), ((), ())),
        preferred_element_type=jnp.float32)
    s3 = s_big.reshape(QB, M, QB)
    eye = (jax.lax.broadcasted_iota(jnp.int32, (QB, 1, QB), 0)
           == jax.lax.broadcasted_iota(jnp.int32, (QB, 1, QB), 2)
           ).astype(jnp.float32)
    s_all = jnp.sum(s3 * eye, axis=2)

    m_i = jax.lax.broadcasted_iota(jnp.int32, (QB, M, M), 1)
    j_i = jax.lax.broadcasted_iota(jnp.int32, (QB, M, M), 2)

    a = r[:, :, None]
    b = r[:, None, :]
    rank2 = jnp.sum(((b > a) | ((b == a) & (j_i < m_i))).astype(jnp.int32),
                    axis=2)
    oh = (rank2[:, :, None] == j_i).astype(jnp.float32)
    sortedv = jnp.sum(oh * r[:, :, None], axis=1)

    ohp = (pre[:, :, None] == j_i).astype(jnp.float32)
    s_perm = jnp.sum(ohp * s_all[:, None, :], axis=2)

    res = (sortedv + s_perm) * jnp.float32(0.5)

    a2 = res[:, :, None]
    b2 = res[:, None, :]
    rank = jnp.sum(((b2 > a2) | ((b2 == a2) & (j_i < m_i))).astype(jnp.int32),
                   axis=2)
    oh2 = (rank[:, :, None] == j_i).astype(jnp.float32)
    reord = jnp.sum(oh2 * rrv.astype(jnp.float32)[:, :, None], axis=1)
    head_ref[...] = reord.astype(jnp.int32)


NW = 32
CH = 200
NCHT = N_DB // CH
CPW = (NCHT + NW - 1) // NW


def _sc_copy_body(ranks_hbm, out_hbm, buf, sem_in, sem_o0, sem_o1):
    wid = jax.lax.axis_index("s") * 2 + jax.lax.axis_index("c")
    sems = (sem_o0, sem_o1)

    def chunk_row(c):
        t = c * NW + wid
        teff = jnp.where(t < NCHT, t, wid)
        return pl.multiple_of(teff * CH, 8)

    pltpu.make_async_copy(
        ranks_hbm.at[pl.ds(chunk_row(0), CH)], buf.at[0], sem_in).start()
    for c in range(CPW):
        cur = c & 1
        nxt = 1 - cur
        row = chunk_row(c)
        pltpu.make_async_copy(
            ranks_hbm.at[pl.ds(row, CH)], buf.at[cur], sem_in).wait()
        pltpu.make_async_copy(
            buf.at[cur], out_hbm.at[pl.ds(row, CH)], sems[cur]).start()
        if c + 1 < CPW:
            if c >= 1:
                prow = chunk_row(c - 1)
                pltpu.make_async_copy(
                    buf.at[nxt], out_hbm.at[pl.ds(prow, CH)],
                    sems[nxt]).wait()
            pltpu.make_async_copy(
                ranks_hbm.at[pl.ds(chunk_row(c + 1), CH)], buf.at[nxt],
                sem_in).start()
    last = (CPW - 1) & 1
    for b, c in ((last, CPW - 1), (1 - last, CPW - 2)):
        pltpu.make_async_copy(
            buf.at[b], out_hbm.at[pl.ds(chunk_row(c), CH)],
            sems[b]).wait()


_sc_copy = functools.partial(
    pl.kernel,
    out_type=jax.ShapeDtypeStruct((N_DB, Q), jnp.int32),
    mesh=plsc.VectorSubcoreMesh(core_axis_name="c", subcore_axis_name="s"),
    scratch_types=[
        pltpu.VMEM((2, CH, Q), jnp.int32),
        pltpu.SemaphoreType.DMA,
        pltpu.SemaphoreType.DMA,
        pltpu.SemaphoreType.DMA,
    ],
)(_sc_copy_body)


def _fix_head_body(head_ref, tail_ref, out_ref):
    del tail_ref
    out_ref[...] = jnp.transpose(head_ref[...], (1, 0))


def _asm_body(head_ref, ranks_ref, out_ref):
    i = pl.program_id(0)

    @pl.when(i == 0)
    def _():
        out_ref[...] = jnp.transpose(head_ref[...], (1, 0))

    @pl.when(i > 0)
    def _():
        out_ref[...] = ranks_ref[...]


def kernel(ranks, rerank_dba_final, res_top1000_dba, ranks_trans_1000_pre,
           x_dba):
    prek = ranks_trans_1000_pre[:, :K]
    head = pl.pallas_call(
        _head_body,
        grid=(Q // QB,),
        in_specs=[
            pl.BlockSpec(memory_space=pltpu.SMEM),
            pl.BlockSpec((QB, M), lambda i: (i, 0)),
            pl.BlockSpec((QB, M), lambda i: (i, 0)),
            pl.BlockSpec((QB, M), lambda i: (i, 0)),
            pl.BlockSpec((QB, M, D), lambda i: (i, 0, 0)),
        ],
        out_specs=pl.BlockSpec((QB, M), lambda i: (i, 0)),
        out_shape=jax.ShapeDtypeStruct((Q, M), jnp.int32),
    )(prek, ranks_trans_1000_pre, res_top1000_dba, rerank_dba_final, x_dba)

    tail = _sc_copy(ranks)
    out = pl.pallas_call(
        _fix_head_body,
        grid=(1,),
        in_specs=[
            pl.BlockSpec((Q, M), lambda i: (0, 0)),
            pl.BlockSpec(memory_space=pl.ANY),
        ],
        out_specs=pl.BlockSpec((M, Q), lambda i: (0, 0)),
        out_shape=jax.ShapeDtypeStruct((N_DB, Q), jnp.int32),
        input_output_aliases={1: 0},
    )(head, tail)
    return out
